# Initial kernel scaffold; baseline (speedup 1.0000x reference)
import jax
import jax.numpy as jnp
from jax.experimental import pallas as pl


def kernel(x_embed, prompt_pool, prompt_key, ctx, w_alpha, b_alpha, token_prefix, token_suffix, penalty_factors, train_flag):
    raise NotImplementedError("write your pallas kernel here")


# speedup vs baseline: 1.1546x; 1.1546x over previous
import functools

import jax
import jax.numpy as jnp
from jax.experimental import pallas as pl
from jax.experimental.pallas import tpu as pltpu

POOL = 32
PLEN = 32
NCTX = 32
ED = 768
TOPK = 5
NCLS = 100
SUF = 12
CB = 20
NBLK = NCLS // CB


def _fused_body(x_ref, key_ref, pen_ref, flag_ref, pool_ref, w_ref, b_ref,
                ctx_ref, pre_ref, suf_ref, out_ref, comb_ref):
    pid = pl.program_id(0)

    @pl.when(pid == 0)
    def _route():
        x = jnp.mean(x_ref[...], axis=0)
        key = key_ref[...]
        dots = jnp.sum(key * x[None, :], axis=1)
        inv = jax.lax.rsqrt(jnp.sum(key * key, axis=1))
        s = dots * inv
        s = jnp.where(flag_ref[0, 0] != 0, s * pen_ref[0, :], s)
        si = s[:, None]
        sj = s[None, :]
        ii = jax.lax.broadcasted_iota(jnp.int32, (POOL, POOL), 0)
        jj = jax.lax.broadcasted_iota(jnp.int32, (POOL, POOL), 1)
        beats = (sj > si) | ((sj == si) & (jj < ii))
        rank = jnp.sum(beats.astype(jnp.int32), axis=1)
        mask = (rank < TOPK).astype(jnp.float32)

        pool = pool_ref[...]
        w = w_ref[0, :]
        z = jnp.sum(pool * w[None, None, :], axis=-1) + b_ref[0, 0]
        alpha = 1.0 / (1.0 + jnp.exp(-z))
        wgt = alpha * mask[:, None]
        comb_ref[...] = jnp.sum(wgt[:, :, None] * pool, axis=0)

    out_ref[:, 0:1, :] = pre_ref[...]
    comb = comb_ref[...]
    out_ref[:, 1:1 + PLEN, :] = jnp.broadcast_to(comb[None], (CB, PLEN, ED))
    out_ref[:, 1 + PLEN:1 + PLEN + NCTX, :] = jnp.broadcast_to(
        ctx_ref[...][None], (CB, NCTX, ED))
    out_ref[:, 1 + PLEN + NCTX:, :] = suf_ref[...]


@jax.jit
def _run(x_embed, prompt_pool, prompt_key, ctx, w_alpha, b_alpha,
         token_prefix, token_suffix, penalty_factors, train_flag):
    pen2 = penalty_factors.reshape(1, POOL)
    flag2 = jnp.asarray(train_flag, jnp.int32).reshape(1, 1)
    b2 = b_alpha.reshape(1, 1)
    grid = (NBLK,)
    full = lambda *shape: pl.BlockSpec(shape, lambda i: (0,) * len(shape))
    prompts = pl.pallas_call(
        _fused_body,
        grid=grid,
        in_specs=[
            full(16, ED),
            full(POOL, ED),
            full(1, POOL),
            full(1, 1),
            full(POOL, PLEN, ED),
            full(1, ED),
            full(1, 1),
            full(NCTX, ED),
            pl.BlockSpec((CB, 1, ED), lambda i: (i, 0, 0)),
            pl.BlockSpec((CB, SUF, ED), lambda i: (i, 0, 0)),
        ],
        out_specs=pl.BlockSpec((CB, 77, ED), lambda i: (i, 0, 0)),
        out_shape=jax.ShapeDtypeStruct((NCLS, 77, ED), jnp.float32),
        scratch_shapes=[pltpu.VMEM((PLEN, ED), jnp.float32)],
    )(x_embed, prompt_key, pen2, flag2, prompt_pool, w_alpha, b2,
      ctx, token_prefix, token_suffix)
    return prompts


def kernel(x_embed, prompt_pool, prompt_key, ctx, w_alpha, b_alpha,
           token_prefix, token_suffix, penalty_factors, train_flag):
    prompts = _run(x_embed, prompt_pool, prompt_key, ctx, w_alpha, b_alpha,
                   token_prefix, token_suffix, penalty_factors, train_flag)
    return (prompts, prompt_pool, prompt_key)
